# Initial kernel scaffold; baseline (speedup 1.0000x reference)
import jax
import jax.numpy as jnp
from jax.experimental import pallas as pl


def kernel(input_ids, node_embedding, edge_weight, node_weight):
    raise NotImplementedError("write your pallas kernel here")


# speedup vs baseline: 2.3206x; 2.3206x over previous
import functools

import jax
import jax.numpy as jnp
from jax import lax
from jax.experimental import pallas as pl
from jax.experimental.pallas import tpu as pltpu
from jax.experimental.pallas import tpu_sc as plsc

VOCAB = 5000
D = 128
P = 2
B, L = 1024, 50
BL = B * L
K = 2 * P

NC, NS = 2, 16
NT = NC * NS
PER_TILE = BL // NT
CW = 80
NCH = PER_TILE // CW
EPT = K * PER_TILE
ECH = EPT // CW


def _sc_gather_body(emb_hbm, ewt_hbm, nwt_hbm, nidx_hbm, eidx_hbm,
                    e_out, w_out, n_out,
                    nidx_v, eidx_v, rows_v, wsc_v, nsc_v, sem):
    wid = lax.axis_index("s") * NC + lax.axis_index("c")
    pltpu.sync_copy(nidx_hbm.at[wid], nidx_v)
    pltpu.sync_copy(eidx_hbm.at[wid], eidx_v)

    base = wid * PER_TILE

    def nbody(j, carry):
        row0 = base + j * CW
        pltpu.async_copy(emb_hbm.at[nidx_v.at[j]], rows_v, sem).wait()
        pltpu.sync_copy(rows_v, e_out.at[pl.ds(row0, CW)])
        pltpu.async_copy(nwt_hbm.at[nidx_v.at[j]], nsc_v, sem).wait()
        pltpu.sync_copy(nsc_v, n_out.at[pl.ds(row0, CW)])
        return carry

    lax.fori_loop(0, NCH, nbody, 0)

    ebase = wid * EPT

    def ebody(j, carry):
        pltpu.async_copy(ewt_hbm.at[eidx_v.at[j]], wsc_v, sem).wait()
        pltpu.sync_copy(wsc_v, w_out.at[pl.ds(ebase + j * CW, CW)])
        return carry

    lax.fori_loop(0, ECH, ebody, 0)


@functools.cache
def _sc_gather():
    return pl.kernel(
        _sc_gather_body,
        mesh=plsc.VectorSubcoreMesh(core_axis_name="c", subcore_axis_name="s",
                                    num_cores=NC, num_subcores=NS),
        out_type=(
            jax.ShapeDtypeStruct((BL, D), jnp.float32),
            jax.ShapeDtypeStruct((K * BL,), jnp.float32),
            jax.ShapeDtypeStruct((BL,), jnp.float32),
        ),
        scratch_types=[
            pltpu.VMEM((NCH, CW), jnp.int32),
            pltpu.VMEM((ECH, CW), jnp.int32),
            pltpu.VMEM((CW, D), jnp.float32),
            pltpu.VMEM((CW,), jnp.float32),
            pltpu.VMEM((CW,), jnp.float32),
            pltpu.SemaphoreType.DMA,
        ],
    )


BB = 128


def _tc_body(e_ref, w_ref, nw_ref, o_ref):
    E = e_ref[...]
    w = w_ref[...]
    nw = nw_ref[...]
    z1 = jnp.zeros((BB, 1, D), jnp.float32)
    z2 = jnp.zeros((BB, 2, D), jnp.float32)
    ra0 = jnp.concatenate([z2, E[:, : L - 2]], axis=1)
    ra1 = jnp.concatenate([z1, E[:, : L - 1]], axis=1)
    ra2 = jnp.concatenate([E[:, 1:], z1], axis=1)
    ra3 = jnp.concatenate([E[:, 2:], z2], axis=1)
    m = jnp.maximum(
        jnp.maximum(ra0 * w[0][:, :, None], ra1 * w[1][:, :, None]),
        jnp.maximum(ra2 * w[2][:, :, None], ra3 * w[3][:, :, None]),
    )
    nwl = nw[:, :, None]
    o_ref[...] = ((1.0 - nwl) * m + nwl * E).sum(axis=1)


def kernel(input_ids, node_embedding, edge_weight, node_weight):
    ids = input_ids.astype(jnp.int32)
    xp = jnp.pad(ids, ((0, 0), (P, P)))
    nx = jnp.stack([xp[:, P + off: P + off + L]
                    for off in (-2, -1, 1, 2)], axis=0)
    ew_ids = ids[None] * VOCAB + nx
    ew_ids = jnp.where(nx == 0, 0, ew_ids)

    nidx = ids.reshape(NT, NCH, CW)
    eidx = ew_ids.reshape(NT, ECH, CW)
    emb = node_embedding.astype(jnp.float32)
    ewt = edge_weight.reshape(-1).astype(jnp.float32)
    nwt = node_weight.reshape(-1).astype(jnp.float32)

    e_out, w_out, n_out = _sc_gather()(emb, ewt, nwt, nidx, eidx)

    E = e_out.reshape(B, L, D)
    w4 = w_out.reshape(K, B, L)
    nw = n_out.reshape(B, L)

    return pl.pallas_call(
        _tc_body,
        grid=(B // BB,),
        in_specs=[
            pl.BlockSpec((BB, L, D), lambda i: (i, 0, 0)),
            pl.BlockSpec((K, BB, L), lambda i: (0, i, 0)),
            pl.BlockSpec((BB, L), lambda i: (i, 0)),
        ],
        out_specs=pl.BlockSpec((BB, D), lambda i: (i, 0)),
        out_shape=jax.ShapeDtypeStruct((B, D), jnp.float32),
    )(E, w4, nw)
